# Initial kernel scaffold; baseline (speedup 1.0000x reference)
import jax
import jax.numpy as jnp
from jax.experimental import pallas as pl


def kernel(pillar_features, coords):
    raise NotImplementedError("write your pallas kernel here")


# speedup vs baseline: 1.5866x; 1.5866x over previous
import functools

import jax
import jax.numpy as jnp
from jax import lax
from jax.experimental import pallas as pl
from jax.experimental.pallas import tpu as pltpu
from jax.experimental.pallas import tpu_sc as plsc

_NX, _NY, _NZ = 360, 360, 2
_V = _NZ * _NY * _NX
_B = 2
_C = 64
_P = 120000
_PER = _P // _B
_NPAD = 128
_TR = 16384
_VC = 512
_NCK = _TR // _VC
_CB = 4000
_NB = _PER // _CB


def _sc_body(feat, coords, canvas, masks,
             coords_v, inv_v, gath_v, outt_v, mask_v, sem):
    c = lax.axis_index("c")
    t = lax.axis_index("s")
    iota = lax.iota(jnp.int32, 16)
    lo = pl.multiple_of(jnp.minimum(t * _TR, _V - _TR), 128)

    def fill_body(ii, carry):
        inv_v[pl.ds(ii * 16, 16)] = _P + ((iota + ii * 16) & (_NPAD - 1))
        return carry
    lax.fori_loop(0, _TR // 16, fill_body, 0)

    def blk_body(blk, carry):
        base_p = pl.multiple_of(c * _PER + blk * _CB, 8)
        pltpu.sync_copy(coords.at[pl.ds(base_p, _CB)], coords_v)

        def grp_body(g, carry2):
            r = iota + g * 16
            zc = plsc.load_gather(coords_v, [r, jnp.full((16,), 1, jnp.int32)])
            yc = plsc.load_gather(coords_v, [r, jnp.full((16,), 2, jnp.int32)])
            xc = plsc.load_gather(coords_v, [r, jnp.full((16,), 3, jnp.int32)])
            v = zc * (_NY * _NX) + yc * _NX + xc
            m = (v >= lo) & (v < lo + _TR)
            idx = jnp.clip(v - lo, 0, _TR - 1)
            plsc.store_scatter(inv_v, [idx], base_p + g * 16 + iota, mask=m)
            return carry2
        lax.fori_loop(0, _CB // 16, grp_body, 0)
        return carry
    lax.fori_loop(0, _NB, blk_body, 0)

    def chunk_body(kk, carry):
        v0 = pl.multiple_of(lo + kk * _VC, 128)
        hs = []
        for j in range(_VC // 128):
            hs.append(pltpu.async_copy(
                feat.at[inv_v.at[pl.ds(kk * _VC + j * 128, 128)]],
                gath_v.at[pl.ds(j * 128, 128)], sem))
        for h in hs:
            h.wait()

        def tr_body(g, carry2):
            row = iota + g * 16
            for cc in range(_C):
                outt_v[cc, pl.ds(g * 16, 16)] = plsc.load_gather(
                    gath_v, [row, jnp.full((16,), cc, jnp.int32)])
            return carry2
        lax.fori_loop(0, _VC // 16, tr_body, 0)

        for o in range(_VC // 16):
            vv = inv_v[pl.ds(kk * _VC + o * 16, 16)]
            mask_v[pl.ds(o * 16, 16)] = jnp.where(
                vv < _P, jnp.int32(1), jnp.int32(0))
        pltpu.sync_copy(mask_v, masks.at[pl.ds(c * _V + v0, _VC)])
        pltpu.sync_copy(outt_v, canvas.at[pl.ds(c * _C, _C), pl.ds(v0, _VC)])
        return carry
    lax.fori_loop(0, _NCK, chunk_body, 0)


@functools.partial(
    pl.kernel,
    out_type=(
        jax.ShapeDtypeStruct((_B * _C, _V), jnp.float32),
        jax.ShapeDtypeStruct((_B * _V,), jnp.int32),
    ),
    mesh=plsc.VectorSubcoreMesh(core_axis_name="c", subcore_axis_name="s"),
    compiler_params=pltpu.CompilerParams(
        needs_layout_passes=False, use_tc_tiling_on_sc=False),
    scratch_types=[
        pltpu.VMEM((_CB, 4), jnp.int32),
        pltpu.VMEM((_TR,), jnp.int32),
        pltpu.VMEM((_VC, _C), jnp.float32),
        pltpu.VMEM((_C, _VC), jnp.float32),
        pltpu.VMEM((_VC,), jnp.int32),
        pltpu.SemaphoreType.DMA,
    ],
)
def _scatter3d_sc(feat, coords, canvas, masks, *scratch):
    _sc_body(feat, coords, canvas, masks, *scratch)


def kernel(pillar_features, coords):
    featpad = jnp.concatenate(
        [pillar_features, jnp.zeros((_NPAD, _C), jnp.float32)], axis=0)
    canvas, masks_i32 = _scatter3d_sc(featpad, coords)
    batch_spatial_features = canvas.reshape(_B, _C * _NZ, _NY, _NX)
    masks = masks_i32.reshape(_B, _V) != 0
    return batch_spatial_features, masks


# speedup vs baseline: 1.6791x; 1.0584x over previous
import functools

import jax
import jax.numpy as jnp
from jax import lax
from jax.experimental import pallas as pl
from jax.experimental.pallas import tpu as pltpu
from jax.experimental.pallas import tpu_sc as plsc

_NX, _NY, _NZ = 360, 360, 2
_V = _NZ * _NY * _NX
_B = 2
_C = 64
_P = 120000
_PER = _P // _B
_NPAD = 128
_TR = 16896
_VC = 384
_NCK = _TR // _VC
_CB = 1200
_NB = _PER // _CB
_LO_MAX = _V - _TR


def _sc_body(feat, coords, canvas, masks, coords_v,
             inv_v, gath0, gath1, outt0, outt1, mask0, mask1,
             gsem0, gsem1, wsem0, wsem1):
    c = lax.axis_index("c")
    t = lax.axis_index("s")
    iota = lax.iota(jnp.int32, 16)
    lo = pl.multiple_of(jnp.minimum(t * _TR, _LO_MAX), 128)
    gath = (gath0, gath1)
    outt = (outt0, outt1)
    maskb = (mask0, mask1)
    gsem = (gsem0, gsem1)
    wsem = (wsem0, wsem1)

    def fill_body(ii, carry):
        inv_v[pl.ds(ii * 16, 16)] = _P + ((iota + ii * 16) & (_NPAD - 1))
        return carry
    lax.fori_loop(0, _TR // 16, fill_body, 0)

    def blk_body(blk, carry):
        base_p = pl.multiple_of(c * _PER + blk * _CB, 8)
        pltpu.sync_copy(coords.at[pl.ds(base_p, _CB)], coords_v)

        def grp_body(g, carry2):
            r = iota + g * 16
            zc = plsc.load_gather(coords_v, [r, jnp.full((16,), 1, jnp.int32)])
            yc = plsc.load_gather(coords_v, [r, jnp.full((16,), 2, jnp.int32)])
            xc = plsc.load_gather(coords_v, [r, jnp.full((16,), 3, jnp.int32)])
            v = zc * (_NY * _NX) + yc * _NX + xc
            m = (v >= lo) & (v < lo + _TR)
            idx = jnp.clip(v - lo, 0, _TR - 1)
            plsc.store_scatter(inv_v, [idx], base_p + g * 16 + iota, mask=m)
            return carry2
        lax.fori_loop(0, _CB // 16, grp_body, 0)
        return carry
    lax.fori_loop(0, _NB, blk_body, 0)

    def fire_gather(kk, b):
        for j in range(_VC // 128):
            pltpu.async_copy(
                feat.at[inv_v.at[pl.ds(kk * _VC + j * 128, 128)]],
                gath[b].at[pl.ds(j * 128, 128)], gsem[b])

    def drain_gather(b):
        for j in range(_VC // 128):
            pltpu.make_async_copy(
                feat.at[inv_v.at[pl.ds(j * 128, 128)]],
                gath[b].at[pl.ds(j * 128, 128)], gsem[b]).wait()

    def drain_writes(b):
        pltpu.make_async_copy(
            outt[b], canvas.at[pl.ds(c * _C, _C), pl.ds(0, _VC)],
            wsem[b]).wait()
        pltpu.make_async_copy(
            maskb[b], masks.at[pl.ds(0, _VC)], wsem[b]).wait()

    fire_gather(0, 0)

    def pair_body(i, carry):
        for b in range(2):
            kk = i * 2 + b
            v0 = pl.multiple_of(lo + kk * _VC, 128)

            @pl.when(kk + 1 < _NCK)
            def _():
                fire_gather(kk + 1, b ^ 1)
            drain_gather(b)

            @pl.when(kk >= 2)
            def _():
                drain_writes(b)

            def tr_body(g, carry2):
                row = iota + g * 16
                for cc in range(_C):
                    outt[b][cc, pl.ds(g * 16, 16)] = plsc.load_gather(
                        gath[b], [row, jnp.full((16,), cc, jnp.int32)])
                return carry2
            lax.fori_loop(0, _VC // 16, tr_body, 0)

            for o in range(_VC // 16):
                vv = inv_v[pl.ds(kk * _VC + o * 16, 16)]
                maskb[b][pl.ds(o * 16, 16)] = jnp.where(
                    vv < _P, jnp.int32(1), jnp.int32(0))

            pltpu.async_copy(
                outt[b], canvas.at[pl.ds(c * _C, _C), pl.ds(v0, _VC)],
                wsem[b])
            pltpu.async_copy(
                maskb[b], masks.at[pl.ds(c * _V + v0, _VC)], wsem[b])
        return carry
    lax.fori_loop(0, _NCK // 2, pair_body, 0)
    drain_writes(0)
    drain_writes(1)


@functools.partial(
    pl.kernel,
    out_type=(
        jax.ShapeDtypeStruct((_B * _C, _V), jnp.float32),
        jax.ShapeDtypeStruct((_B * _V,), jnp.int32),
    ),
    mesh=plsc.VectorSubcoreMesh(core_axis_name="c", subcore_axis_name="s"),
    compiler_params=pltpu.CompilerParams(
        needs_layout_passes=False, use_tc_tiling_on_sc=False),
    scratch_types=[
        pltpu.VMEM((_CB, 4), jnp.int32),
        pltpu.VMEM((_TR,), jnp.int32),
        pltpu.VMEM((_VC, _C), jnp.float32),
        pltpu.VMEM((_VC, _C), jnp.float32),
        pltpu.VMEM((_C, _VC), jnp.float32),
        pltpu.VMEM((_C, _VC), jnp.float32),
        pltpu.VMEM((_VC,), jnp.int32),
        pltpu.VMEM((_VC,), jnp.int32),
        pltpu.SemaphoreType.DMA,
        pltpu.SemaphoreType.DMA,
        pltpu.SemaphoreType.DMA,
        pltpu.SemaphoreType.DMA,
    ],
)
def _scatter3d_sc(feat, coords, canvas, masks, *scratch):
    _sc_body(feat, coords, canvas, masks, *scratch)


def kernel(pillar_features, coords):
    featpad = jnp.concatenate(
        [pillar_features, jnp.zeros((_NPAD, _C), jnp.float32)], axis=0)
    canvas, masks_i32 = _scatter3d_sc(featpad, coords)
    batch_spatial_features = canvas.reshape(_B, _C * _NZ, _NY, _NX)
    masks = masks_i32.reshape(_B, _V) != 0
    return batch_spatial_features, masks


# speedup vs baseline: 1.6818x; 1.0016x over previous
import functools

import jax
import jax.numpy as jnp
from jax import lax
from jax.experimental import pallas as pl
from jax.experimental.pallas import tpu as pltpu
from jax.experimental.pallas import tpu_sc as plsc

_NX, _NY, _NZ = 360, 360, 2
_V = _NZ * _NY * _NX
_B = 2
_C = 64
_P = 120000
_PER = _P // _B
_NPAD = 128
_TR = 16896
_VC = 384
_NCK = _TR // _VC
_CB = 1200
_NB = _PER // _CB
_LO_MAX = _V - _TR


def _sc_body(feat, coords, canvas, masks, coords_v,
             inv_v, gath0, gath1, outt0, outt1, mask0, mask1,
             gsem0, gsem1, wsem0, wsem1):
    c = lax.axis_index("c")
    t = lax.axis_index("s")
    iota = lax.iota(jnp.int32, 16)
    lo = pl.multiple_of(jnp.minimum(t * _TR, _LO_MAX), 128)
    gath = (gath0, gath1)
    outt = (outt0, outt1)
    maskb = (mask0, mask1)
    gsem = (gsem0, gsem1)
    wsem = (wsem0, wsem1)

    with jax.named_scope("ph_fill"):
        def fill_body(ii, carry):
            inv_v[pl.ds(ii * 16, 16)] = _P + ((iota + ii * 16) & (_NPAD - 1))
            return carry
        lax.fori_loop(0, _TR // 16, fill_body, 0)

    def blk_body(blk, carry):
      with jax.named_scope("ph_scan"):
        base_p = pl.multiple_of(c * _PER + blk * _CB, 8)
        pltpu.sync_copy(coords.at[pl.ds(base_p, _CB)], coords_v)

        def grp_body(g, carry2):
            r = iota + g * 16
            zc = plsc.load_gather(coords_v, [r, jnp.full((16,), 1, jnp.int32)])
            yc = plsc.load_gather(coords_v, [r, jnp.full((16,), 2, jnp.int32)])
            xc = plsc.load_gather(coords_v, [r, jnp.full((16,), 3, jnp.int32)])
            v = zc * (_NY * _NX) + yc * _NX + xc
            m = (v >= lo) & (v < lo + _TR)
            idx = jnp.clip(v - lo, 0, _TR - 1)
            plsc.store_scatter(inv_v, [idx], base_p + g * 16 + iota, mask=m)
            return carry2
        lax.fori_loop(0, _CB // 16, grp_body, 0)
        return carry
    lax.fori_loop(0, _NB, blk_body, 0)

    def fire_gather(kk, b):
        for j in range(_VC // 128):
            pltpu.async_copy(
                feat.at[inv_v.at[pl.ds(kk * _VC + j * 128, 128)]],
                gath[b].at[pl.ds(j * 128, 128)], gsem[b])

    def drain_gather(b):
        for j in range(_VC // 128):
            pltpu.make_async_copy(
                feat.at[inv_v.at[pl.ds(j * 128, 128)]],
                gath[b].at[pl.ds(j * 128, 128)], gsem[b]).wait()

    def drain_writes(b):
        pltpu.make_async_copy(
            outt[b], canvas.at[pl.ds(c * _C, _C), pl.ds(0, _VC)],
            wsem[b]).wait()
        pltpu.make_async_copy(
            maskb[b], masks.at[pl.ds(0, _VC)], wsem[b]).wait()

    fire_gather(0, 0)

    def pair_body(i, carry):
        for b in range(2):
            kk = i * 2 + b
            v0 = pl.multiple_of(lo + kk * _VC, 128)

            @pl.when(kk + 1 < _NCK)
            def _():
                fire_gather(kk + 1, b ^ 1)
            with jax.named_scope("ph_gwait"):
                drain_gather(b)

                @pl.when(kk >= 2)
                def _():
                    drain_writes(b)

            with jax.named_scope("ph_tr"):
                def tr_body(g, carry2):
                    row = iota + g * 16
                    for cc in range(_C):
                        outt[b][cc, pl.ds(g * 16, 16)] = plsc.load_gather(
                            gath[b], [row, jnp.full((16,), cc, jnp.int32)])
                    return carry2
                lax.fori_loop(0, _VC // 16, tr_body, 0)

            with jax.named_scope("ph_wr"):
                for o in range(_VC // 16):
                    vv = inv_v[pl.ds(kk * _VC + o * 16, 16)]
                    maskb[b][pl.ds(o * 16, 16)] = jnp.where(
                        vv < _P, jnp.int32(1), jnp.int32(0))

                pltpu.async_copy(
                    outt[b], canvas.at[pl.ds(c * _C, _C), pl.ds(v0, _VC)],
                    wsem[b])
                pltpu.async_copy(
                    maskb[b], masks.at[pl.ds(c * _V + v0, _VC)], wsem[b])
        return carry
    lax.fori_loop(0, _NCK // 2, pair_body, 0)
    drain_writes(0)
    drain_writes(1)


@functools.partial(
    pl.kernel,
    out_type=(
        jax.ShapeDtypeStruct((_B * _C, _V), jnp.float32),
        jax.ShapeDtypeStruct((_B * _V,), jnp.int32),
    ),
    mesh=plsc.VectorSubcoreMesh(core_axis_name="c", subcore_axis_name="s"),
    compiler_params=pltpu.CompilerParams(
        needs_layout_passes=False, use_tc_tiling_on_sc=False),
    scratch_types=[
        pltpu.VMEM((_CB, 4), jnp.int32),
        pltpu.VMEM((_TR,), jnp.int32),
        pltpu.VMEM((_VC, _C), jnp.float32),
        pltpu.VMEM((_VC, _C), jnp.float32),
        pltpu.VMEM((_C, _VC), jnp.float32),
        pltpu.VMEM((_C, _VC), jnp.float32),
        pltpu.VMEM((_VC,), jnp.int32),
        pltpu.VMEM((_VC,), jnp.int32),
        pltpu.SemaphoreType.DMA,
        pltpu.SemaphoreType.DMA,
        pltpu.SemaphoreType.DMA,
        pltpu.SemaphoreType.DMA,
    ],
)
def _scatter3d_sc(feat, coords, canvas, masks, *scratch):
    _sc_body(feat, coords, canvas, masks, *scratch)


def kernel(pillar_features, coords):
    featpad = jnp.concatenate(
        [pillar_features, jnp.zeros((_NPAD, _C), jnp.float32)], axis=0)
    canvas, masks_i32 = _scatter3d_sc(featpad, coords)
    batch_spatial_features = canvas.reshape(_B, _C * _NZ, _NY, _NX)
    masks = masks_i32.reshape(_B, _V) != 0
    return batch_spatial_features, masks


# speedup vs baseline: 2.1753x; 1.2935x over previous
import functools

import jax
import jax.numpy as jnp
from jax import lax
from jax.experimental import pallas as pl
from jax.experimental.pallas import tpu as pltpu
from jax.experimental.pallas import tpu_sc as plsc

_NX, _NY, _NZ = 360, 360, 2
_V = _NZ * _NY * _NX
_B = 2
_C = 64
_P = 120000
_PER = _P // _B
_NPAD = 128
_TR = 16896
_VC = 384
_NCK = _TR // _VC
_CB = 1200
_NB = _PER // _CB
_LO_MAX = _V - _TR


def _sc_body(feat, coords, canvas, masks, coords_v,
             inv_v, gath0, gath1, outt0, outt1, mask0, mask1,
             gsem0, gsem1, wsem0, wsem1):
    c = lax.axis_index("c")
    t = lax.axis_index("s")
    iota = lax.iota(jnp.int32, 16)
    lo = pl.multiple_of(jnp.minimum(t * _TR, _LO_MAX), 128)
    gath = (gath0, gath1)
    outt = (outt0, outt1)
    maskb = (mask0, mask1)
    gsem = (gsem0, gsem1)
    wsem = (wsem0, wsem1)

    with jax.named_scope("ph_fill"):
        def fill_body(ii, carry):
            inv_v[pl.ds(ii * 16, 16)] = _P + ((iota + ii * 16) & (_NPAD - 1))
            return carry
        lax.fori_loop(0, _TR // 16, fill_body, 0)

    def blk_body(blk, carry):
      with jax.named_scope("ph_scan"):
        base_p = pl.multiple_of(c * _PER + blk * _CB, 8)
        pltpu.sync_copy(coords.at[pl.ds(base_p, _CB)], coords_v)

        @plsc.parallel_loop(0, _CB // 16, unroll=2)
        def _(g):
            r = iota + g * 16
            zc = plsc.load_gather(coords_v, [r, jnp.full((16,), 1, jnp.int32)])
            yc = plsc.load_gather(coords_v, [r, jnp.full((16,), 2, jnp.int32)])
            xc = plsc.load_gather(coords_v, [r, jnp.full((16,), 3, jnp.int32)])
            v = zc * (_NY * _NX) + yc * _NX + xc
            m = (v >= lo) & (v < lo + _TR)
            idx = jnp.clip(v - lo, 0, _TR - 1)
            plsc.store_scatter(inv_v, [idx], base_p + g * 16 + iota, mask=m)
        return carry
    lax.fori_loop(0, _NB, blk_body, 0)

    def fire_gather(kk, b):
        for j in range(_VC // 128):
            pltpu.async_copy(
                feat.at[inv_v.at[pl.ds(kk * _VC + j * 128, 128)]],
                gath[b].at[pl.ds(j * 128, 128)], gsem[b])

    def drain_gather(b):
        for j in range(_VC // 128):
            pltpu.make_async_copy(
                feat.at[inv_v.at[pl.ds(j * 128, 128)]],
                gath[b].at[pl.ds(j * 128, 128)], gsem[b]).wait()

    def drain_writes(b):
        pltpu.make_async_copy(
            outt[b], canvas.at[pl.ds(c * _C, _C), pl.ds(0, _VC)],
            wsem[b]).wait()
        pltpu.make_async_copy(
            maskb[b], masks.at[pl.ds(0, _VC)], wsem[b]).wait()

    fire_gather(0, 0)

    def pair_body(i, carry):
        for b in range(2):
            kk = i * 2 + b
            v0 = pl.multiple_of(lo + kk * _VC, 128)

            @pl.when(kk + 1 < _NCK)
            def _():
                fire_gather(kk + 1, b ^ 1)
            with jax.named_scope("ph_gwait"):
                drain_gather(b)

                @pl.when(kk >= 2)
                def _():
                    drain_writes(b)

            with jax.named_scope("ph_tr"):
                @plsc.parallel_loop(0, _VC // 16, unroll=2)
                def _(g):
                    row = iota + g * 16
                    for cc in range(_C):
                        outt[b][cc, pl.ds(g * 16, 16)] = plsc.load_gather(
                            gath[b], [row, jnp.full((16,), cc, jnp.int32)])

            with jax.named_scope("ph_wr"):
                for o in range(_VC // 16):
                    vv = inv_v[pl.ds(kk * _VC + o * 16, 16)]
                    maskb[b][pl.ds(o * 16, 16)] = jnp.where(
                        vv < _P, jnp.int32(1), jnp.int32(0))

                pltpu.async_copy(
                    outt[b], canvas.at[pl.ds(c * _C, _C), pl.ds(v0, _VC)],
                    wsem[b])
                pltpu.async_copy(
                    maskb[b], masks.at[pl.ds(c * _V + v0, _VC)], wsem[b])
        return carry
    lax.fori_loop(0, _NCK // 2, pair_body, 0)
    drain_writes(0)
    drain_writes(1)


@functools.partial(
    pl.kernel,
    out_type=(
        jax.ShapeDtypeStruct((_B * _C, _V), jnp.float32),
        jax.ShapeDtypeStruct((_B * _V,), jnp.int32),
    ),
    mesh=plsc.VectorSubcoreMesh(core_axis_name="c", subcore_axis_name="s"),
    compiler_params=pltpu.CompilerParams(
        needs_layout_passes=False, use_tc_tiling_on_sc=False),
    scratch_types=[
        pltpu.VMEM((_CB, 4), jnp.int32),
        pltpu.VMEM((_TR,), jnp.int32),
        pltpu.VMEM((_VC, _C), jnp.float32),
        pltpu.VMEM((_VC, _C), jnp.float32),
        pltpu.VMEM((_C, _VC), jnp.float32),
        pltpu.VMEM((_C, _VC), jnp.float32),
        pltpu.VMEM((_VC,), jnp.int32),
        pltpu.VMEM((_VC,), jnp.int32),
        pltpu.SemaphoreType.DMA,
        pltpu.SemaphoreType.DMA,
        pltpu.SemaphoreType.DMA,
        pltpu.SemaphoreType.DMA,
    ],
)
def _scatter3d_sc(feat, coords, canvas, masks, *scratch):
    _sc_body(feat, coords, canvas, masks, *scratch)


def kernel(pillar_features, coords):
    featpad = jnp.concatenate(
        [pillar_features, jnp.zeros((_NPAD, _C), jnp.float32)], axis=0)
    canvas, masks_i32 = _scatter3d_sc(featpad, coords)
    batch_spatial_features = canvas.reshape(_B, _C * _NZ, _NY, _NX)
    masks = masks_i32.reshape(_B, _V) != 0
    return batch_spatial_features, masks


# speedup vs baseline: 2.7205x; 1.2506x over previous
import functools

import jax
import jax.numpy as jnp
from jax import lax
from jax.experimental import pallas as pl
from jax.experimental.pallas import tpu as pltpu
from jax.experimental.pallas import tpu_sc as plsc

_NX, _NY, _NZ = 360, 360, 2
_V = _NZ * _NY * _NX
_B = 2
_C = 64
_P = 120000
_PER = _P // _B
_NPAD = 128
_TR = 16896
_VC = 384
_NCK = _TR // _VC
_CB = 1200
_NB = _PER // _CB
_LO_MAX = _V - _TR


def _sc_body(feat, coords, canvas, masks, coords_v,
             inv_v, gath0, gath1, outt0, outt1, mask0, mask1,
             gsem0, gsem1, wsem0, wsem1):
    c = lax.axis_index("c")
    t = lax.axis_index("s")
    iota = lax.iota(jnp.int32, 16)
    lo = pl.multiple_of(jnp.minimum(t * _TR, _LO_MAX), 128)
    gath = (gath0, gath1)
    outt = (outt0, outt1)
    maskb = (mask0, mask1)
    gsem = (gsem0, gsem1)
    wsem = (wsem0, wsem1)

    with jax.named_scope("ph_fill"):
        def fill_body(ii, carry):
            inv_v[pl.ds(ii * 16, 16)] = _P + ((iota + ii * 16) & (_NPAD - 1))
            return carry
        lax.fori_loop(0, _TR // 16, fill_body, 0)

    def blk_body(blk, carry):
      with jax.named_scope("ph_scan"):
        base_p = pl.multiple_of(c * _PER + blk * _CB, 8)
        pltpu.sync_copy(coords.at[:, pl.ds(base_p, _CB)], coords_v)

        @plsc.parallel_loop(0, _CB // 16, unroll=2)
        def _(g):
            zc = coords_v[0, pl.ds(g * 16, 16)]
            yc = coords_v[1, pl.ds(g * 16, 16)]
            xc = coords_v[2, pl.ds(g * 16, 16)]
            v = zc * (_NY * _NX) + yc * _NX + xc
            m = (v >= lo) & (v < lo + _TR)
            idx = jnp.clip(v - lo, 0, _TR - 1)
            plsc.store_scatter(inv_v, [idx], base_p + g * 16 + iota, mask=m)
        return carry
    lax.fori_loop(0, _NB, blk_body, 0)

    def fire_gather(kk, b):
        for j in range(_VC // 128):
            pltpu.async_copy(
                feat.at[inv_v.at[pl.ds(kk * _VC + j * 128, 128)]],
                gath[b].at[pl.ds(j * 128, 128)], gsem[b])

    def drain_gather(b):
        for j in range(_VC // 128):
            pltpu.make_async_copy(
                feat.at[inv_v.at[pl.ds(j * 128, 128)]],
                gath[b].at[pl.ds(j * 128, 128)], gsem[b]).wait()

    def drain_writes(b):
        pltpu.make_async_copy(
            outt[b].at[:, pl.ds(0, _VC)],
            canvas.at[pl.ds(c * _C, _C), pl.ds(0, _VC)],
            wsem[b]).wait()
        pltpu.make_async_copy(
            maskb[b], masks.at[pl.ds(0, _VC)], wsem[b]).wait()

    fire_gather(0, 0)

    def pair_body(i, carry):
        for b in range(2):
            kk = i * 2 + b
            v0 = pl.multiple_of(lo + kk * _VC, 128)

            @pl.when(kk + 1 < _NCK)
            def _():
                fire_gather(kk + 1, b ^ 1)
            with jax.named_scope("ph_gwait"):
                drain_gather(b)

                @pl.when(kk >= 2)
                def _():
                    drain_writes(b)

            with jax.named_scope("ph_tr"):
                @plsc.parallel_loop(0, _VC, unroll=4)
                def _(vi):
                    vv = jnp.full((16,), 0, jnp.int32) + vi
                    for q in range(_C // 16):
                        plsc.store_scatter(
                            outt[b], [iota + q * 16, vv],
                            gath[b][vi, pl.ds(q * 16, 16)])

            with jax.named_scope("ph_wr"):
                for o in range(_VC // 16):
                    vv = inv_v[pl.ds(kk * _VC + o * 16, 16)]
                    maskb[b][pl.ds(o * 16, 16)] = jnp.where(
                        vv < _P, jnp.int32(1), jnp.int32(0))

                pltpu.async_copy(
                    outt[b].at[:, pl.ds(0, _VC)],
                    canvas.at[pl.ds(c * _C, _C), pl.ds(v0, _VC)],
                    wsem[b])
                pltpu.async_copy(
                    maskb[b], masks.at[pl.ds(c * _V + v0, _VC)], wsem[b])
        return carry
    lax.fori_loop(0, _NCK // 2, pair_body, 0)
    drain_writes(0)
    drain_writes(1)


@functools.partial(
    pl.kernel,
    out_type=(
        jax.ShapeDtypeStruct((_B * _C, _V), jnp.float32),
        jax.ShapeDtypeStruct((_B * _V,), jnp.int32),
    ),
    mesh=plsc.VectorSubcoreMesh(core_axis_name="c", subcore_axis_name="s"),
    compiler_params=pltpu.CompilerParams(
        needs_layout_passes=False, use_tc_tiling_on_sc=False),
    scratch_types=[
        pltpu.VMEM((3, _CB), jnp.int32),
        pltpu.VMEM((_TR,), jnp.int32),
        pltpu.VMEM((_VC, _C), jnp.float32),
        pltpu.VMEM((_VC, _C), jnp.float32),
        pltpu.VMEM((_C, _VC + 1), jnp.float32),
        pltpu.VMEM((_C, _VC + 1), jnp.float32),
        pltpu.VMEM((_VC,), jnp.int32),
        pltpu.VMEM((_VC,), jnp.int32),
        pltpu.SemaphoreType.DMA,
        pltpu.SemaphoreType.DMA,
        pltpu.SemaphoreType.DMA,
        pltpu.SemaphoreType.DMA,
    ],
)
def _scatter3d_sc(feat, coords, canvas, masks, *scratch):
    _sc_body(feat, coords, canvas, masks, *scratch)


def kernel(pillar_features, coords):
    featpad = jnp.concatenate(
        [pillar_features, jnp.zeros((_NPAD, _C), jnp.float32)], axis=0)
    zyx = coords[:, 1:4].T
    canvas, masks_i32 = _scatter3d_sc(featpad, zyx)
    batch_spatial_features = canvas.reshape(_B, _C * _NZ, _NY, _NX)
    masks = masks_i32.reshape(_B, _V) != 0
    return batch_spatial_features, masks


# speedup vs baseline: 4.5963x; 1.6895x over previous
import functools

import jax
import jax.numpy as jnp
from jax import lax
from jax.experimental import pallas as pl
from jax.experimental.pallas import tpu as pltpu
from jax.experimental.pallas import tpu_sc as plsc

_NX, _NY, _NZ = 360, 360, 2
_V = _NZ * _NY * _NX
_B = 2
_C = 64
_P = 120000
_PER = _P // _B
_TR = 16896
_VC = 384
_NCK = _TR // _VC
_CB = 1200
_NB = _PER // _CB
_LO_MAX = _V - _TR


def _sc_body(feat, coords, canvas, masks, coords_v,
             inv_v, gath0, gath1, outt0, outt1, mask0, mask1,
             gsem0, gsem1, wsem0, wsem1):
    c = lax.axis_index("c")
    t = lax.axis_index("s")
    iota = lax.iota(jnp.int32, 16)
    lo = pl.multiple_of(jnp.minimum(t * _TR, _LO_MAX), 128)
    gath = (gath0, gath1)
    outt = (outt0, outt1)
    maskb = (mask0, mask1)
    gsem = (gsem0, gsem1)
    wsem = (wsem0, wsem1)

    with jax.named_scope("ph_fill"):
        def fill_body(ii, carry):
            inv_v[pl.ds(ii * 16, 16)] = jnp.full((16,), -1, jnp.int32)
            return carry
        lax.fori_loop(0, _TR // 16, fill_body, 0)

    def blk_body(blk, carry):
      with jax.named_scope("ph_scan"):
        base_p = pl.multiple_of(c * _PER + blk * _CB, 8)
        pltpu.sync_copy(coords.at[:, pl.ds(base_p, _CB)], coords_v)

        @plsc.parallel_loop(0, _CB // 16, unroll=2)
        def _(g):
            zc = coords_v[0, pl.ds(g * 16, 16)]
            yc = coords_v[1, pl.ds(g * 16, 16)]
            xc = coords_v[2, pl.ds(g * 16, 16)]
            v = zc * (_NY * _NX) + yc * _NX + xc
            m = (v >= lo) & (v < lo + _TR)
            idx = jnp.clip(v - lo, 0, _TR - 1)
            plsc.store_scatter(inv_v, [idx], base_p + g * 16 + iota, mask=m)
        return carry
    lax.fori_loop(0, _NB, blk_body, 0)

    def fire_gather(kk, b):
        @plsc.parallel_loop(0, _VC, unroll=4)
        def _(vi):
            for q in range(_C // 16):
                gath[b][vi, pl.ds(q * 16, 16)] = jnp.zeros((16,), jnp.float32)
        for j in range(_VC // 128):
            idx = plsc.Indices(
                inv_v.at[pl.ds(kk * _VC + j * 128, 128)], ignored_value=-1)
            pltpu.async_copy(
                feat.at[idx], gath[b].at[pl.ds(j * 128, 128)], gsem[b])

    def drain_gather(b):
        for j in range(_VC // 128):
            idx = plsc.Indices(
                inv_v.at[pl.ds(j * 128, 128)], ignored_value=-1)
            pltpu.make_async_copy(
                feat.at[idx], gath[b].at[pl.ds(j * 128, 128)], gsem[b]).wait()

    def drain_writes(b):
        pltpu.make_async_copy(
            outt[b].at[:, pl.ds(0, _VC)],
            canvas.at[pl.ds(c * _C, _C), pl.ds(0, _VC)],
            wsem[b]).wait()
        pltpu.make_async_copy(
            maskb[b], masks.at[pl.ds(0, _VC)], wsem[b]).wait()

    fire_gather(0, 0)

    def pair_body(i, carry):
        for b in range(2):
            kk = i * 2 + b
            v0 = pl.multiple_of(lo + kk * _VC, 128)

            @pl.when(kk + 1 < _NCK)
            def _():
                fire_gather(kk + 1, b ^ 1)
            with jax.named_scope("ph_gwait"):
                drain_gather(b)

                @pl.when(kk >= 2)
                def _():
                    drain_writes(b)

            with jax.named_scope("ph_tr"):
                @plsc.parallel_loop(0, _VC, unroll=4)
                def _(vi):
                    vv = jnp.full((16,), 0, jnp.int32) + vi
                    for q in range(_C // 16):
                        plsc.store_scatter(
                            outt[b], [iota + q * 16, vv],
                            gath[b][vi, pl.ds(q * 16, 16)])

            with jax.named_scope("ph_wr"):
                for o in range(_VC // 16):
                    vv = inv_v[pl.ds(kk * _VC + o * 16, 16)]
                    maskb[b][pl.ds(o * 16, 16)] = jnp.where(
                        vv >= 0, jnp.int32(1), jnp.int32(0))

                pltpu.async_copy(
                    outt[b].at[:, pl.ds(0, _VC)],
                    canvas.at[pl.ds(c * _C, _C), pl.ds(v0, _VC)],
                    wsem[b])
                pltpu.async_copy(
                    maskb[b], masks.at[pl.ds(c * _V + v0, _VC)], wsem[b])
        return carry
    lax.fori_loop(0, _NCK // 2, pair_body, 0)
    drain_writes(0)
    drain_writes(1)


@functools.partial(
    pl.kernel,
    out_type=(
        jax.ShapeDtypeStruct((_B * _C, _V), jnp.float32),
        jax.ShapeDtypeStruct((_B * _V,), jnp.int32),
    ),
    mesh=plsc.VectorSubcoreMesh(core_axis_name="c", subcore_axis_name="s"),
    compiler_params=pltpu.CompilerParams(
        needs_layout_passes=False, use_tc_tiling_on_sc=False),
    scratch_types=[
        pltpu.VMEM((3, _CB), jnp.int32),
        pltpu.VMEM((_TR,), jnp.int32),
        pltpu.VMEM((_VC, _C), jnp.float32),
        pltpu.VMEM((_VC, _C), jnp.float32),
        pltpu.VMEM((_C, _VC + 1), jnp.float32),
        pltpu.VMEM((_C, _VC + 1), jnp.float32),
        pltpu.VMEM((_VC,), jnp.int32),
        pltpu.VMEM((_VC,), jnp.int32),
        pltpu.SemaphoreType.DMA,
        pltpu.SemaphoreType.DMA,
        pltpu.SemaphoreType.DMA,
        pltpu.SemaphoreType.DMA,
    ],
)
def _scatter3d_sc(feat, coords, canvas, masks, *scratch):
    _sc_body(feat, coords, canvas, masks, *scratch)


def kernel(pillar_features, coords):
    zyx = coords[:, 1:4].T
    canvas, masks_i32 = _scatter3d_sc(pillar_features, zyx)
    batch_spatial_features = canvas.reshape(_B, _C * _NZ, _NY, _NX)
    masks = masks_i32.reshape(_B, _V) != 0
    return batch_spatial_features, masks
